# baseline (speedup 1.0000x reference)
import jax
import jax.numpy as jnp
from jax import lax
from jax.experimental import pallas as pl
from jax.experimental.pallas import tpu as pltpu
from jax.experimental.pallas import tpu_sc as plsc

_B = 16384
_E = 16
_NC = 2
_NS = 16
_NW = _NC * _NS
_BPW = _B // _NW
_CH = 128
_NCH = _BPW // _CH


def _sc_body(uidx_hbm, bidx_hbm, uemb_hbm, ubias_hbm, bemb_hbm, bbias_hbm,
             partial_hbm, bsum_hbm,
             uidx_v, bidx_v, urows_v, brows_v, ubias_v, bbias_v, bsum_v,
             acc_v, sem):
    wid = lax.axis_index("s") * _NC + lax.axis_index("c")

    pltpu.sync_copy(uidx_hbm.at[pl.ds(wid * _NCH, _NCH)], uidx_v)
    pltpu.sync_copy(bidx_hbm.at[pl.ds(wid * _NCH, _NCH)], bidx_v)

    copies = []
    for k in range(_NCH):
        sl = pl.ds(k * _CH, _CH)
        copies.append(pltpu.async_copy(uemb_hbm.at[uidx_v.at[k]],
                                       urows_v.at[sl], sem))
        copies.append(pltpu.async_copy(bemb_hbm.at[bidx_v.at[k]],
                                       brows_v.at[sl], sem))
        copies.append(pltpu.async_copy(ubias_hbm.at[uidx_v.at[k]],
                                       ubias_v.at[sl], sem))
        copies.append(pltpu.async_copy(bbias_hbm.at[bidx_v.at[k]],
                                       bbias_v.at[sl], sem))
    for c in copies:
        c.wait()

    def dot_body(i, acc):
        for j in range(4):
            r = i * 4 + j
            acc = acc + urows_v[r] * brows_v[r]
        return acc

    acc = lax.fori_loop(0, _BPW // 4, dot_body, jnp.zeros((16,), jnp.float32))
    acc_v[...] = acc
    pltpu.sync_copy(acc_v, partial_hbm.at[pl.ds(wid * 16, 16)])

    def bias_body(i, carry):
        sl = pl.ds(i * 16, 16)
        bsum_v[sl] = ubias_v[sl] + bbias_v[sl]
        return carry

    lax.fori_loop(0, _BPW // 16, bias_body, 0)
    pltpu.sync_copy(bsum_v, bsum_hbm.at[pl.ds(wid * _BPW, _BPW)])


_sc_call = pl.kernel(
    _sc_body,
    out_type=(jax.ShapeDtypeStruct((_NW * 16,), jnp.float32),
              jax.ShapeDtypeStruct((_B,), jnp.float32)),
    mesh=plsc.VectorSubcoreMesh(core_axis_name="c", subcore_axis_name="s"),
    scratch_types=[
        pltpu.VMEM((_NCH, _CH), jnp.int32),
        pltpu.VMEM((_NCH, _CH), jnp.int32),
        pltpu.VMEM((_BPW, _E), jnp.float32),
        pltpu.VMEM((_BPW, _E), jnp.float32),
        pltpu.VMEM((_BPW,), jnp.float32),
        pltpu.VMEM((_BPW,), jnp.float32),
        pltpu.VMEM((_BPW,), jnp.float32),
        pltpu.VMEM((16,), jnp.float32),
        pltpu.SemaphoreType.DMA,
    ],
    compiler_params=pltpu.CompilerParams(use_tc_tiling_on_sc=False),
)


def _fin_body(p_ref, bs_ref, o_ref):
    s = jnp.sum(p_ref[...])
    o_ref[...] = jax.nn.sigmoid(bs_ref[...] + s)


def kernel(inputs, user_embedding, user_bias, book_embedding, book_bias):
    idx = inputs.astype(jnp.int32)
    uidx = idx[:, 0].reshape(_B // _CH, _CH)
    bidx = idx[:, 1].reshape(_B // _CH, _CH)
    partials, bsum = _sc_call(uidx, bidx, user_embedding,
                              user_bias.reshape(-1), book_embedding,
                              book_bias.reshape(-1))
    out = pl.pallas_call(
        _fin_body,
        out_shape=jax.ShapeDtypeStruct((_B // 128, 128), jnp.float32),
    )(partials.reshape(4, 128), bsum.reshape(_B // 128, 128))
    return out.reshape(_B, 1)


# speedup vs baseline: 2.9587x; 2.9587x over previous
import jax
import jax.numpy as jnp
from jax import lax
from jax.experimental import pallas as pl
from jax.experimental.pallas import tpu as pltpu
from jax.experimental.pallas import tpu_sc as plsc

_B = 16384
_E = 16
_NC = 2
_NS = 16
_NW = _NC * _NS
_BPW = _B // _NW
_CH = 128
_NCH = _BPW // _CH
_RING = 16
_NG = _BPW // _RING


def _gather_table(tab_hbm, idx_v, blk_v, dst_v, ring_sem):
    rows = lax.iota(jnp.int32, _E)

    def body(g, carry):
        @pl.when(g > 0)
        def _extract():
            gg = g - 1
            vec = idx_v[gg // 8, pl.ds((gg % 8) * _E, _E)]
            for j in range(_RING):
                pltpu.make_async_copy(tab_hbm.at[:, pl.ds(0, _CH)],
                                      blk_v.at[j], ring_sem.at[j]).wait()
                lanes = jnp.full((_E,), vec[j] & 127, jnp.int32)
                slot = jnp.full((_E,), j, jnp.int32)
                col = plsc.load_gather(blk_v, [slot, rows, lanes])
                dst_v[pl.ds((gg * _RING + j) * _E, _E)] = col

        @pl.when(g < _NG)
        def _issue():
            vec = idx_v[g // 8, pl.ds((g % 8) * _E, _E)]
            for j in range(_RING):
                base = pl.multiple_of((vec[j] >> 7) * _CH, _CH)
                pltpu.async_copy(tab_hbm.at[:, pl.ds(base, _CH)],
                                 blk_v.at[j], ring_sem.at[j])

        return carry

    lax.fori_loop(0, _NG + 1, body, 0)


def _sc_body(uidx_hbm, bidx_hbm, uembt_hbm, ubias_hbm, bembt_hbm, bbias_hbm,
             partial_hbm, bsum_hbm,
             uidx_v, bidx_v, blk_v, uloc_v, bloc_v, ubias_v, bbias_v, bsum_v,
             acc_v, gsem, ring_sem):
    wid = lax.axis_index("s") * _NC + lax.axis_index("c")

    pltpu.sync_copy(uidx_hbm.at[pl.ds(wid * _NCH, _NCH)], uidx_v)
    pltpu.sync_copy(bidx_hbm.at[pl.ds(wid * _NCH, _NCH)], bidx_v)

    bias_copies = []
    for k in range(_NCH):
        sl = pl.ds(k * _CH, _CH)
        bias_copies.append(pltpu.async_copy(ubias_hbm.at[uidx_v.at[k]],
                                            ubias_v.at[sl], gsem))
        bias_copies.append(pltpu.async_copy(bbias_hbm.at[bidx_v.at[k]],
                                            bbias_v.at[sl], gsem))

    _gather_table(uembt_hbm, uidx_v, blk_v, uloc_v, ring_sem)
    _gather_table(bembt_hbm, bidx_v, blk_v, bloc_v, ring_sem)

    for c in bias_copies:
        c.wait()

    def dot_body(c, acc):
        sl = pl.ds(c * _E, _E)
        return acc + uloc_v[sl] * bloc_v[sl]

    acc = lax.fori_loop(0, _BPW, dot_body, jnp.zeros((_E,), jnp.float32))
    acc_v[...] = acc
    pltpu.sync_copy(acc_v, partial_hbm.at[pl.ds(wid * _E, _E)])

    def bias_body(i, carry):
        sl = pl.ds(i * _E, _E)
        bsum_v[sl] = ubias_v[sl] + bbias_v[sl]
        return carry

    lax.fori_loop(0, _BPW // _E, bias_body, 0)
    pltpu.sync_copy(bsum_v, bsum_hbm.at[pl.ds(wid * _BPW, _BPW)])


_sc_call = pl.kernel(
    _sc_body,
    out_type=(jax.ShapeDtypeStruct((_NW * _E,), jnp.float32),
              jax.ShapeDtypeStruct((_B,), jnp.float32)),
    mesh=plsc.VectorSubcoreMesh(core_axis_name="c", subcore_axis_name="s"),
    scratch_types=[
        pltpu.VMEM((_NCH, _CH), jnp.int32),
        pltpu.VMEM((_NCH, _CH), jnp.int32),
        pltpu.VMEM((_RING, _E, _CH), jnp.float32),
        pltpu.VMEM((_BPW * _E,), jnp.float32),
        pltpu.VMEM((_BPW * _E,), jnp.float32),
        pltpu.VMEM((_BPW,), jnp.float32),
        pltpu.VMEM((_BPW,), jnp.float32),
        pltpu.VMEM((_BPW,), jnp.float32),
        pltpu.VMEM((_E,), jnp.float32),
        pltpu.SemaphoreType.DMA,
        pltpu.SemaphoreType.DMA((_RING,)),
    ],
    compiler_params=pltpu.CompilerParams(use_tc_tiling_on_sc=True,
                                         needs_layout_passes=False),
)


def _fin_body(p_ref, bs_ref, o_ref):
    s = jnp.sum(p_ref[...])
    o_ref[...] = jax.nn.sigmoid(bs_ref[...] + s)


def kernel(inputs, user_embedding, user_bias, book_embedding, book_bias):
    idx = inputs.astype(jnp.int32)
    uidx = idx[:, 0].reshape(_B // _CH, _CH)
    bidx = idx[:, 1].reshape(_B // _CH, _CH)
    partials, bsum = _sc_call(uidx, bidx, user_embedding.T,
                              user_bias.reshape(-1), book_embedding.T,
                              book_bias.reshape(-1))
    out = pl.pallas_call(
        _fin_body,
        out_shape=jax.ShapeDtypeStruct((_B // 128, 128), jnp.float32),
    )(partials.reshape(4, 128), bsum.reshape(_B // 128, 128))
    return out.reshape(_B, 1)
